# baseline (speedup 1.0000x reference)
import functools

import jax
import jax.numpy as jnp
from jax import lax
from jax.experimental import pallas as pl
from jax.experimental.pallas import tpu as pltpu
from jax.experimental.pallas import tpu_sc as plsc

B, C, N = 4, 128, 1024
K = 20
OUT = 256
KP = 32
NEG = -3.0e38

NW = 32
PW = N // NW
CH = 4
NCH = PW // CH


def _prep_body(x_ref, w_ref, y1_ref, z_ref, idx_ref):
    x = x_ref[...]
    dnt = (((0,), (0,)), ((), ()))
    g = lax.dot_general(x, x, dnt,
                        preferred_element_type=jnp.float32)
    xx = jnp.sum(x * x, axis=0, keepdims=True)
    d = (2.0 * g - jnp.transpose(xx)) - xx

    col = lax.broadcasted_iota(jnp.int32, (N, N), 1)
    lanek = lax.broadcasted_iota(jnp.int32, (N, KP), 1)
    idxacc = jnp.zeros((N, KP), jnp.int32)
    for k in range(K):
        m = jnp.max(d, axis=1, keepdims=True)
        cand = jnp.where(d == m, col, N)
        am = jnp.min(cand, axis=1, keepdims=True)
        idxacc = jnp.where(lanek == k, am, idxacc)
        d = jnp.where(col == am, NEG, d)
    idx_ref[...] = idxacc

    w1 = w_ref[:, :C]
    wd = w_ref[:, C:] - w1
    dnp = (((0,), (1,)), ((), ()))
    y1_ref[...] = lax.dot_general(x, w1, dnp,
                                  preferred_element_type=jnp.float32,
                                  precision=lax.Precision.HIGHEST)
    z_ref[...] = lax.dot_general(x, wd, dnp,
                                 preferred_element_type=jnp.float32,
                                 precision=lax.Precision.HIGHEST)


_prep1 = pl.pallas_call(
    _prep_body,
    out_shape=[
        jax.ShapeDtypeStruct((N, OUT), jnp.float32),
        jax.ShapeDtypeStruct((N, OUT), jnp.float32),
        jax.ShapeDtypeStruct((N, KP), jnp.int32),
    ],
)


def _gather_max(y1b, idxb, zb):
    mesh = plsc.VectorSubcoreMesh(core_axis_name="c", subcore_axis_name="s")

    @functools.partial(
        pl.kernel,
        out_type=jax.ShapeDtypeStruct((N, OUT), jnp.float32),
        mesh=mesh,
        scratch_types=[
            pltpu.VMEM((PW * K,), jnp.int32),
            pltpu.VMEM((PW, OUT), jnp.float32),
            pltpu.VMEM((CH * K, OUT), jnp.float32),
            pltpu.VMEM((CH * K, OUT), jnp.float32),
            pltpu.SemaphoreType.DMA,
            pltpu.SemaphoreType.DMA,
        ],
    )
    def body(y1_hbm, idx_hbm, z_hbm, out_hbm,
             idx_v, out_v, rows0, rows1, sem0, sem1):
        wid = lax.axis_index("s") * 2 + lax.axis_index("c")
        row0w = wid * PW
        pltpu.sync_copy(idx_hbm.at[pl.ds(row0w * K, PW * K)], idx_v)
        pltpu.sync_copy(z_hbm.at[pl.ds(row0w, PW)], out_v)

        def gather(ch, rows, sem):
            return pltpu.async_copy(
                y1_hbm.at[idx_v.at[pl.ds(ch * CH * K, CH * K)]], rows, sem)

        def compute(ch, rows):
            def pair_body(p, carry):
                for c in range(OUT // 16):
                    sl = pl.ds(c * 16, 16)
                    acc = rows[p * K, sl]
                    for kk in range(1, K):
                        acc = jnp.maximum(acc, rows[p * K + kk, sl])
                    q = ch * CH + p
                    out_v[q, sl] = out_v[q, sl] + acc
                return carry
            lax.fori_loop(0, CH, pair_body, 0)

        def cp_wait(sem, rows):
            pltpu.make_async_copy(y1_hbm.at[idx_v.at[pl.ds(0, CH * K)]],
                                  rows, sem).wait()

        gather(0, rows0, sem0)

        def two_chunks(i, carry):
            ch0 = i * 2
            gather(ch0 + 1, rows1, sem1)
            cp_wait(sem0, rows0)
            compute(ch0, rows0)
            @pl.when(i < NCH // 2 - 1)
            def _():
                gather(ch0 + 2, rows0, sem0)
            cp_wait(sem1, rows1)
            compute(ch0 + 1, rows1)
            return carry

        lax.fori_loop(0, NCH // 2, two_chunks, 0)
        pltpu.sync_copy(out_v, out_hbm.at[pl.ds(row0w, PW)])

    return body(y1b, idxb, zb)


def kernel(x, W):
    outs = []
    for b in range(B):
        y1, z, idx = _prep1(x[b], W)
        outs.append(_gather_max(y1, idx[:, :K].reshape(N * K), z))
    out = jnp.stack(outs)
    return jnp.transpose(out, (0, 2, 1))


# speedup vs baseline: 1.0126x; 1.0126x over previous
import functools

import jax
import jax.numpy as jnp
from jax import lax
from jax.experimental import pallas as pl
from jax.experimental.pallas import tpu as pltpu
from jax.experimental.pallas import tpu_sc as plsc

B, C, N = 4, 128, 1024
K = 20
OUT = 256
KP = 32
NEG = -3.0e38

NW = 32
PW = N // NW
CH = 4
NCH = PW // CH


def _prep_body(x_ref, w_ref, y1_ref, z_ref, idx_ref):
    x = x_ref[...]
    dnt = (((0,), (0,)), ((), ()))
    g = lax.dot_general(x, x, dnt,
                        preferred_element_type=jnp.float32)
    xx = jnp.sum(x * x, axis=0, keepdims=True)
    d = (2.0 * g - jnp.transpose(xx)) - xx

    col = lax.broadcasted_iota(jnp.int32, (N, N), 1)
    lanek = lax.broadcasted_iota(jnp.int32, (N, KP), 1)
    idxacc = jnp.zeros((N, KP), jnp.int32)
    m = jnp.max(d, axis=1, keepdims=True)
    for k in range(K):
        cand = jnp.where(d == m, col, N)
        am = jnp.min(cand, axis=1, keepdims=True)
        idxacc = jnp.where(lanek == k, am, idxacc)
        if k < K - 1:
            d = jnp.where(col == am, NEG, d)
            m = jnp.max(d, axis=1, keepdims=True)
    idx_ref[...] = idxacc[:, :K]

    w1 = w_ref[:, :C]
    wd = w_ref[:, C:] - w1
    dnp = (((0,), (1,)), ((), ()))
    y1_ref[...] = lax.dot_general(x, w1, dnp,
                                  preferred_element_type=jnp.float32)
    z_ref[...] = lax.dot_general(x, wd, dnp,
                                 preferred_element_type=jnp.float32)


_prep1 = pl.pallas_call(
    _prep_body,
    out_shape=[
        jax.ShapeDtypeStruct((N, OUT), jnp.float32),
        jax.ShapeDtypeStruct((N, OUT), jnp.float32),
        jax.ShapeDtypeStruct((N, K), jnp.int32),
    ],
)


def _gather_max(y1b, idxb, zb):
    mesh = plsc.VectorSubcoreMesh(core_axis_name="c", subcore_axis_name="s")

    @functools.partial(
        pl.kernel,
        out_type=jax.ShapeDtypeStruct((N, OUT), jnp.float32),
        mesh=mesh,
        scratch_types=[
            pltpu.VMEM((PW * K,), jnp.int32),
            pltpu.VMEM((PW, OUT), jnp.float32),
            pltpu.VMEM((CH * K, OUT), jnp.float32),
            pltpu.VMEM((CH * K, OUT), jnp.float32),
            pltpu.SemaphoreType.DMA,
            pltpu.SemaphoreType.DMA,
        ],
    )
    def body(y1_hbm, idx_hbm, z_hbm, out_hbm,
             idx_v, out_v, rows0, rows1, sem0, sem1):
        wid = lax.axis_index("s") * 2 + lax.axis_index("c")
        row0w = wid * PW
        pltpu.sync_copy(idx_hbm.at[pl.ds(row0w * K, PW * K)], idx_v)
        pltpu.sync_copy(z_hbm.at[pl.ds(row0w, PW)], out_v)

        def gather(ch, rows, sem):
            return pltpu.async_copy(
                y1_hbm.at[idx_v.at[pl.ds(ch * CH * K, CH * K)]], rows, sem)

        def compute(ch, rows):
            def pair_body(p, carry):
                for c in range(OUT // 16):
                    sl = pl.ds(c * 16, 16)
                    acc = rows[p * K, sl]
                    for kk in range(1, K):
                        acc = jnp.maximum(acc, rows[p * K + kk, sl])
                    q = ch * CH + p
                    out_v[q, sl] = out_v[q, sl] + acc
                return carry
            lax.fori_loop(0, CH, pair_body, 0)

        def cp_wait(sem, rows):
            pltpu.make_async_copy(y1_hbm.at[idx_v.at[pl.ds(0, CH * K)]],
                                  rows, sem).wait()

        gather(0, rows0, sem0)

        def two_chunks(i, carry):
            ch0 = i * 2
            gather(ch0 + 1, rows1, sem1)
            cp_wait(sem0, rows0)
            compute(ch0, rows0)
            @pl.when(i < NCH // 2 - 1)
            def _():
                gather(ch0 + 2, rows0, sem0)
            cp_wait(sem1, rows1)
            compute(ch0 + 1, rows1)
            return carry

        lax.fori_loop(0, NCH // 2, two_chunks, 0)
        pltpu.sync_copy(out_v, out_hbm.at[pl.ds(row0w, PW)])

    return body(y1b, idxb, zb)


def kernel(x, W):
    outs = []
    for b in range(B):
        y1, z, idx = _prep1(x[b], W)
        outs.append(_gather_max(y1, idx.reshape(N * K), z))
    out = jnp.stack(outs)
    return jnp.transpose(out, (0, 2, 1))


# speedup vs baseline: 1.0129x; 1.0003x over previous
import functools

import jax
import jax.numpy as jnp
from jax import lax
from jax.experimental import pallas as pl
from jax.experimental.pallas import tpu as pltpu
from jax.experimental.pallas import tpu_sc as plsc

B, C, N = 4, 128, 1024
K = 20
OUT = 256
KP = 32
NEG = -3.0e38

NW = 32
PW = N // NW
CH = 4
NCH = PW // CH


def _prep_body(x_ref, w_ref, y1_ref, z_ref, idx_ref):
    x = x_ref[...]
    dnt = (((0,), (0,)), ((), ()))
    g = lax.dot_general(x, x, dnt,
                        preferred_element_type=jnp.float32)
    xx = jnp.sum(x * x, axis=0, keepdims=True)
    d = (2.0 * g - jnp.transpose(xx)) - xx

    col = lax.broadcasted_iota(jnp.int32, (N, N), 1)
    lanek = lax.broadcasted_iota(jnp.int32, (N, KP), 1)
    idxacc = jnp.zeros((N, KP), jnp.int32)
    m = jnp.max(d, axis=1, keepdims=True)
    for k in range(K):
        cand = jnp.where(d == m, col, N)
        am = jnp.min(cand, axis=1, keepdims=True)
        idxacc = jnp.where(lanek == k, am, idxacc)
        if k < K - 1:
            d = jnp.where(col == am, NEG, d)
            m = jnp.max(d, axis=1, keepdims=True)
    idx_ref[...] = idxacc

    w1 = w_ref[:, :C]
    wd = w_ref[:, C:] - w1
    dnp = (((0,), (1,)), ((), ()))
    y1_ref[...] = lax.dot_general(x, w1, dnp,
                                  preferred_element_type=jnp.float32)
    z_ref[...] = lax.dot_general(x, wd, dnp,
                                 preferred_element_type=jnp.float32)


_prep1 = pl.pallas_call(
    _prep_body,
    out_shape=[
        jax.ShapeDtypeStruct((N, OUT), jnp.float32),
        jax.ShapeDtypeStruct((N, OUT), jnp.float32),
        jax.ShapeDtypeStruct((N, KP), jnp.int32),
    ],
)


def _gather_max(y1b, idxb, zb):
    mesh = plsc.VectorSubcoreMesh(core_axis_name="c", subcore_axis_name="s")

    @functools.partial(
        pl.kernel,
        out_type=jax.ShapeDtypeStruct((N, OUT), jnp.float32),
        mesh=mesh,
        scratch_types=[
            pltpu.VMEM((PW * K,), jnp.int32),
            pltpu.VMEM((PW, OUT), jnp.float32),
            pltpu.VMEM((CH * K, OUT), jnp.float32),
            pltpu.VMEM((CH * K, OUT), jnp.float32),
            pltpu.SemaphoreType.DMA,
            pltpu.SemaphoreType.DMA,
        ],
    )
    def body(y1_hbm, idx_hbm, z_hbm, out_hbm,
             idx_v, out_v, rows0, rows1, sem0, sem1):
        wid = lax.axis_index("s") * 2 + lax.axis_index("c")
        row0w = wid * PW
        pltpu.sync_copy(idx_hbm.at[pl.ds(row0w * K, PW * K)], idx_v)
        pltpu.sync_copy(z_hbm.at[pl.ds(row0w, PW)], out_v)

        def gather(ch, rows, sem):
            return pltpu.async_copy(
                y1_hbm.at[idx_v.at[pl.ds(ch * CH * K, CH * K)]], rows, sem)

        def compute(ch, rows):
            def pair_body(p, carry):
                for c in range(OUT // 16):
                    sl = pl.ds(c * 16, 16)
                    acc = rows[p * K, sl]
                    for kk in range(1, K):
                        acc = jnp.maximum(acc, rows[p * K + kk, sl])
                    q = ch * CH + p
                    out_v[q, sl] = out_v[q, sl] + acc
                return carry
            lax.fori_loop(0, CH, pair_body, 0)

        def cp_wait(sem, rows):
            pltpu.make_async_copy(y1_hbm.at[idx_v.at[pl.ds(0, CH * K)]],
                                  rows, sem).wait()

        gather(0, rows0, sem0)

        def two_chunks(i, carry):
            ch0 = i * 2
            gather(ch0 + 1, rows1, sem1)
            cp_wait(sem0, rows0)
            compute(ch0, rows0)
            @pl.when(i < NCH // 2 - 1)
            def _():
                gather(ch0 + 2, rows0, sem0)
            cp_wait(sem1, rows1)
            compute(ch0 + 1, rows1)
            return carry

        lax.fori_loop(0, NCH // 2, two_chunks, 0)
        pltpu.sync_copy(out_v, out_hbm.at[pl.ds(row0w, PW)])

    return body(y1b, idxb, zb)


def kernel(x, W):
    outs = []
    for b in range(B):
        y1, z, idx = _prep1(x[b], W)
        o = _gather_max(y1, idx[:, :K].reshape(N * K), z)
        outs.append(jnp.transpose(o))
    return jnp.stack(outs)


# speedup vs baseline: 1.0800x; 1.0662x over previous
import functools

import jax
import jax.numpy as jnp
from jax import lax
from jax.experimental import pallas as pl
from jax.experimental.pallas import tpu as pltpu
from jax.experimental.pallas import tpu_sc as plsc

B, C, N = 4, 128, 1024
K = 20
OUT = 256
KP = 32
NEG = -3.0e38

NG = 2
GB = B // NG
RG = GB * N

NW = 32
PW = RG // NW
CH = 4
NCH = PW // CH


def _prep_body(x_ref, w_ref, y1_ref, z_ref, idx_ref):
    g = pl.program_id(0)
    x = x_ref[...]
    dnt = (((0,), (0,)), ((), ()))
    gram = lax.dot_general(x, x, dnt,
                           preferred_element_type=jnp.float32)
    xx = jnp.sum(x * x, axis=0, keepdims=True)
    d = (2.0 * gram - jnp.transpose(xx)) - xx

    col = lax.broadcasted_iota(jnp.int32, (N, N), 1)
    lanek = lax.broadcasted_iota(jnp.int32, (N, KP), 1)
    base = g * N
    idxacc = jnp.full((N, KP), base, jnp.int32)
    m = jnp.max(d, axis=1, keepdims=True)
    for k in range(K):
        cand = jnp.where(d == m, col, N)
        am = jnp.min(cand, axis=1, keepdims=True)
        idxacc = jnp.where(lanek == k, am + base, idxacc)
        if k < K - 1:
            d = jnp.where(col == am, NEG, d)
            m = jnp.max(d, axis=1, keepdims=True)
    idx_ref[...] = idxacc

    w1 = w_ref[:, :C]
    wd = w_ref[:, C:] - w1
    dnp = (((0,), (1,)), ((), ()))
    y1_ref[...] = lax.dot_general(x, w1, dnp,
                                  preferred_element_type=jnp.float32)
    z_ref[...] = lax.dot_general(x, wd, dnp,
                                 preferred_element_type=jnp.float32)


def _prep(x, w, grp):
    return pl.pallas_call(
        _prep_body,
        grid=(GB,),
        in_specs=[
            pl.BlockSpec((None, C, N), lambda g: (grp * GB + g, 0, 0)),
            pl.BlockSpec((OUT, 2 * C), lambda g: (0, 0)),
        ],
        out_specs=[
            pl.BlockSpec((None, N, OUT), lambda g: (g, 0, 0)),
            pl.BlockSpec((None, N, OUT), lambda g: (g, 0, 0)),
            pl.BlockSpec((None, N, KP), lambda g: (g, 0, 0)),
        ],
        out_shape=[
            jax.ShapeDtypeStruct((GB, N, OUT), jnp.float32),
            jax.ShapeDtypeStruct((GB, N, OUT), jnp.float32),
            jax.ShapeDtypeStruct((GB, N, KP), jnp.int32),
        ],
    )(x, w)


def _gather_max(y1g, idxg, zg):
    mesh = plsc.VectorSubcoreMesh(core_axis_name="c", subcore_axis_name="s")

    @functools.partial(
        pl.kernel,
        out_type=jax.ShapeDtypeStruct((RG, OUT), jnp.float32),
        mesh=mesh,
        scratch_types=[
            pltpu.VMEM((PW * K,), jnp.int32),
            pltpu.VMEM((PW, OUT), jnp.float32),
            pltpu.VMEM((CH * K, OUT), jnp.float32),
            pltpu.VMEM((CH * K, OUT), jnp.float32),
            pltpu.SemaphoreType.DMA,
            pltpu.SemaphoreType.DMA,
        ],
    )
    def body(y1_hbm, idx_hbm, z_hbm, out_hbm,
             idx_v, out_v, rows0, rows1, sem0, sem1):
        wid = lax.axis_index("s") * 2 + lax.axis_index("c")
        row0w = wid * PW
        pltpu.sync_copy(idx_hbm.at[pl.ds(row0w * K, PW * K)], idx_v)
        pltpu.sync_copy(z_hbm.at[pl.ds(row0w, PW)], out_v)

        def gather(ch, rows, sem):
            return pltpu.async_copy(
                y1_hbm.at[idx_v.at[pl.ds(ch * CH * K, CH * K)]], rows, sem)

        def compute(ch, rows):
            def pair_body(p, carry):
                for c in range(OUT // 16):
                    sl = pl.ds(c * 16, 16)
                    acc = rows[p * K, sl]
                    for kk in range(1, K):
                        acc = jnp.maximum(acc, rows[p * K + kk, sl])
                    q = ch * CH + p
                    out_v[q, sl] = out_v[q, sl] + acc
                return carry
            lax.fori_loop(0, CH, pair_body, 0)

        def cp_wait(sem, rows):
            pltpu.make_async_copy(y1_hbm.at[idx_v.at[pl.ds(0, CH * K)]],
                                  rows, sem).wait()

        gather(0, rows0, sem0)

        def two_chunks(i, carry):
            ch0 = i * 2
            gather(ch0 + 1, rows1, sem1)
            cp_wait(sem0, rows0)
            compute(ch0, rows0)
            @pl.when(i < NCH // 2 - 1)
            def _():
                gather(ch0 + 2, rows0, sem0)
            cp_wait(sem1, rows1)
            compute(ch0 + 1, rows1)
            return carry

        lax.fori_loop(0, NCH // 2, two_chunks, 0)
        pltpu.sync_copy(out_v, out_hbm.at[pl.ds(row0w, PW)])

    return body(y1g, idxg, zg)


def kernel(x, W):
    outs = []
    for grp in range(NG):
        y1, z, idx = _prep(x, W, grp)
        o = _gather_max(y1.reshape(RG, OUT),
                        idx[:, :, :K].reshape(RG * K),
                        z.reshape(RG, OUT))
        outs.append(o.reshape(GB, N, OUT))
    out = jnp.concatenate(outs)
    return jnp.transpose(out, (0, 2, 1))
